# baseline (speedup 1.0000x reference)
import jax
import jax.numpy as jnp
from jax.experimental import pallas as pl
from jax.experimental.pallas import tpu as pltpu

_REINIT_P = 0.05
_N, _R, _C = 10000, 250, 100
_B = 128
_K = 16
_CH = _N // _K


def _body(idx_s, w_s, u_s, buf, noise, ns_row, idx_col, idx_row, u_col,
          out, new_buf, steps_out, new_ns_out, sem_a, sem_b, sem_c):
    for k in range(_K):
        pltpu.make_async_copy(buf.at[pl.ds(k * _CH, _CH)],
                              new_buf.at[pl.ds(k * _CH, _CH)], sem_a).start()

    def issue_b(b, carry):
        reinit = u_s[b] < _REINIT_P

        @pl.when(reinit)
        def _():
            pltpu.make_async_copy(noise.at[pl.ds(b, 1)],
                                  out.at[pl.ds(b, 1)], sem_b).start()

        @pl.when(jnp.logical_not(reinit))
        def _():
            pltpu.make_async_copy(buf.at[pl.ds(idx_s[b], 1)],
                                  out.at[pl.ds(b, 1)], sem_b).start()

        return carry

    jax.lax.fori_loop(0, _B, issue_b, 0)

    ns = ns_row[...]
    ic = idx_col[...]
    ir = idx_row[...]
    rc = u_col[...] < _REINIT_P
    col_ids = jax.lax.broadcasted_iota(jnp.int32, (_B, _N), 1)
    onehot = ic == col_ids
    g = jnp.sum(jnp.where(onehot, ns, 0.0), axis=1, keepdims=True)
    steps = jnp.where(rc, 0.0, g)
    steps_out[...] = steps
    bi = jax.lax.broadcasted_iota(jnp.int32, (_B, _B), 0)
    bj = jax.lax.broadcasted_iota(jnp.int32, (_B, _B), 1)
    later_same = (ic == ir) & (bj > bi)
    winner = jnp.logical_not(jnp.any(later_same, axis=1, keepdims=True))
    sc_mask = onehot & winner
    contrib = jnp.sum(jnp.where(sc_mask, steps, 0.0), axis=0, keepdims=True)
    written = jnp.any(sc_mask, axis=0, keepdims=True)
    new_ns_out[...] = jnp.where(written, contrib, ns)

    def wait_b(b, carry):
        pltpu.make_async_copy(noise.at[pl.ds(0, 1)],
                              out.at[pl.ds(0, 1)], sem_b).wait()
        return carry

    jax.lax.fori_loop(0, _B, wait_b, 0)
    for k in range(_K):
        pltpu.make_async_copy(buf.at[pl.ds(k * _CH, _CH)],
                              new_buf.at[pl.ds(k * _CH, _CH)], sem_a).wait()

    def issue_c(b, carry):
        cond = (u_s[b] < _REINIT_P) & (w_s[b] == b)

        @pl.when(cond)
        def _():
            pltpu.make_async_copy(noise.at[pl.ds(b, 1)],
                                  new_buf.at[pl.ds(idx_s[b], 1)], sem_c).start()

        return carry

    jax.lax.fori_loop(0, _B, issue_c, 0)

    def wait_c(b, carry):
        cond = (u_s[b] < _REINIT_P) & (w_s[b] == b)

        @pl.when(cond)
        def _():
            pltpu.make_async_copy(noise.at[pl.ds(b, 1)],
                                  new_buf.at[pl.ds(idx_s[b], 1)], sem_c).wait()

        return carry

    jax.lax.fori_loop(0, _B, wait_c, 0)


def kernel(buffer, buffer_numsteps, noise, u, idx):
    idx = idx.astype(jnp.int32)
    eq = idx[:, None] == idx[None, :]
    w = jnp.max(jnp.where(eq, jnp.arange(_B, dtype=jnp.int32)[None, :], -1), axis=1)

    smem = pltpu.MemorySpace.SMEM
    hbm = pltpu.MemorySpace.HBM
    out, new_buffer, steps, new_ns = pl.pallas_call(
        _body,
        in_specs=[
            pl.BlockSpec(memory_space=smem),
            pl.BlockSpec(memory_space=smem),
            pl.BlockSpec(memory_space=smem),
            pl.BlockSpec(memory_space=hbm),
            pl.BlockSpec(memory_space=hbm),
            pl.BlockSpec((1, _N), lambda: (0, 0)),
            pl.BlockSpec((_B, 1), lambda: (0, 0)),
            pl.BlockSpec((1, _B), lambda: (0, 0)),
            pl.BlockSpec((_B, 1), lambda: (0, 0)),
        ],
        out_specs=[
            pl.BlockSpec(memory_space=hbm),
            pl.BlockSpec(memory_space=hbm),
            pl.BlockSpec((_B, 1), lambda: (0, 0)),
            pl.BlockSpec((1, _N), lambda: (0, 0)),
        ],
        out_shape=[
            jax.ShapeDtypeStruct((_B, _R, _C), jnp.float32),
            jax.ShapeDtypeStruct((_N, _R, _C), jnp.float32),
            jax.ShapeDtypeStruct((_B, 1), jnp.float32),
            jax.ShapeDtypeStruct((1, _N), jnp.float32),
        ],
        scratch_shapes=[pltpu.SemaphoreType.DMA] * 3,
    )(idx, w, u, buffer, noise, buffer_numsteps.reshape(1, _N),
      idx.reshape(_B, 1), idx.reshape(1, _B), u.reshape(_B, 1))

    return out, steps.reshape(_B), new_buffer, new_ns.reshape(_N)


# speedup vs baseline: 12.8934x; 12.8934x over previous
import jax
import jax.numpy as jnp
from jax.experimental import pallas as pl
from jax.experimental.pallas import tpu as pltpu

_REINIT_P = 0.05
_N, _R, _C = 10000, 250, 100
_B = 128
_Q = 8
_CH = 10
_G = _N // (_Q * _CH)


def _body(idx_s, w_s, u_s, buf, noise, ns_row, idx_col, idx_row, u_col,
          out, new_buf, steps_out, new_ns_out, cp_v, row_v, sem_a, sem_b):
    def copy_group(g, carry):
        base = g * _Q * _CH
        for q in range(_Q):
            pltpu.make_async_copy(buf.at[pl.ds(base + q * _CH, _CH)],
                                  cp_v.at[q], sem_a).start()
        for q in range(_Q):
            pltpu.make_async_copy(buf.at[pl.ds(base + q * _CH, _CH)],
                                  cp_v.at[q], sem_a).wait()
        for q in range(_Q):
            pltpu.make_async_copy(cp_v.at[q],
                                  new_buf.at[pl.ds(base + q * _CH, _CH)],
                                  sem_a).start()
        for q in range(_Q):
            pltpu.make_async_copy(cp_v.at[q],
                                  new_buf.at[pl.ds(base + q * _CH, _CH)],
                                  sem_a).wait()
        return carry

    ns = ns_row[...]
    ic = idx_col[...]
    ir = idx_row[...]
    rc = u_col[...] < _REINIT_P
    col_ids = jax.lax.broadcasted_iota(jnp.int32, (_B, _N), 1)
    onehot = ic == col_ids
    g = jnp.sum(jnp.where(onehot, ns, 0.0), axis=1, keepdims=True)
    steps = jnp.where(rc, 0.0, g)
    steps_out[...] = steps
    bi = jax.lax.broadcasted_iota(jnp.int32, (_B, _B), 0)
    bj = jax.lax.broadcasted_iota(jnp.int32, (_B, _B), 1)
    later_same = (ic == ir) & (bj > bi)
    winner = jnp.logical_not(jnp.any(later_same, axis=1, keepdims=True))
    sc_mask = onehot & winner
    contrib = jnp.sum(jnp.where(sc_mask, steps, 0.0), axis=0, keepdims=True)
    written = jnp.any(sc_mask, axis=0, keepdims=True)
    new_ns_out[...] = jnp.where(written, contrib, ns)

    jax.lax.fori_loop(0, _G, copy_group, 0)

    def gather_group(gg, carry):
        base = gg * _Q
        for q in range(_Q):
            b = base + q
            reinit = u_s[b] < _REINIT_P

            @pl.when(reinit)
            def _():
                pltpu.make_async_copy(noise.at[pl.ds(b, 1)],
                                      row_v.at[q], sem_b).start()

            @pl.when(jnp.logical_not(reinit))
            def _():
                pltpu.make_async_copy(buf.at[pl.ds(idx_s[b], 1)],
                                      row_v.at[q], sem_b).start()

        for q in range(_Q):
            pltpu.make_async_copy(noise.at[pl.ds(0, 1)],
                                  row_v.at[q], sem_b).wait()
        for q in range(_Q):
            b = base + q
            pltpu.make_async_copy(row_v.at[q],
                                  out.at[pl.ds(b, 1)], sem_b).start()
        for q in range(_Q):
            b = base + q
            pltpu.make_async_copy(row_v.at[q],
                                  out.at[pl.ds(b, 1)], sem_b).wait()
        return carry

    jax.lax.fori_loop(0, _B // _Q, gather_group, 0)

    def scatter_one(b, carry):
        cond = (u_s[b] < _REINIT_P) & (w_s[b] == b)

        @pl.when(cond)
        def _():
            pltpu.make_async_copy(noise.at[pl.ds(b, 1)],
                                  row_v.at[0], sem_b).start()
            pltpu.make_async_copy(noise.at[pl.ds(b, 1)],
                                  row_v.at[0], sem_b).wait()
            pltpu.make_async_copy(row_v.at[0],
                                  new_buf.at[pl.ds(idx_s[b], 1)], sem_b).start()
            pltpu.make_async_copy(row_v.at[0],
                                  new_buf.at[pl.ds(idx_s[b], 1)], sem_b).wait()

        return carry

    jax.lax.fori_loop(0, _B, scatter_one, 0)


def kernel(buffer, buffer_numsteps, noise, u, idx):
    idx = idx.astype(jnp.int32)
    eq = idx[:, None] == idx[None, :]
    w = jnp.max(jnp.where(eq, jnp.arange(_B, dtype=jnp.int32)[None, :], -1), axis=1)

    smem = pltpu.MemorySpace.SMEM
    hbm = pltpu.MemorySpace.HBM
    out, new_buffer, steps, new_ns = pl.pallas_call(
        _body,
        in_specs=[
            pl.BlockSpec(memory_space=smem),
            pl.BlockSpec(memory_space=smem),
            pl.BlockSpec(memory_space=smem),
            pl.BlockSpec(memory_space=hbm),
            pl.BlockSpec(memory_space=hbm),
            pl.BlockSpec((1, _N), lambda: (0, 0)),
            pl.BlockSpec((_B, 1), lambda: (0, 0)),
            pl.BlockSpec((1, _B), lambda: (0, 0)),
            pl.BlockSpec((_B, 1), lambda: (0, 0)),
        ],
        out_specs=[
            pl.BlockSpec(memory_space=hbm),
            pl.BlockSpec(memory_space=hbm),
            pl.BlockSpec((_B, 1), lambda: (0, 0)),
            pl.BlockSpec((1, _N), lambda: (0, 0)),
        ],
        out_shape=[
            jax.ShapeDtypeStruct((_B, _R, _C), jnp.float32),
            jax.ShapeDtypeStruct((_N, _R, _C), jnp.float32),
            jax.ShapeDtypeStruct((_B, 1), jnp.float32),
            jax.ShapeDtypeStruct((1, _N), jnp.float32),
        ],
        scratch_shapes=[
            pltpu.VMEM((_Q, _CH, _R, _C), jnp.float32),
            pltpu.VMEM((_Q, 1, _R, _C), jnp.float32),
            pltpu.SemaphoreType.DMA,
            pltpu.SemaphoreType.DMA,
        ],
    )(idx, w, u, buffer, noise, buffer_numsteps.reshape(1, _N),
      idx.reshape(_B, 1), idx.reshape(1, _B), u.reshape(_B, 1))

    return out, steps.reshape(_B), new_buffer, new_ns.reshape(_N)


# speedup vs baseline: 13.5201x; 1.0486x over previous
import jax
import jax.numpy as jnp
from jax.experimental import pallas as pl
from jax.experimental.pallas import tpu as pltpu

_REINIT_P = 0.05
_N, _R, _C = 10000, 250, 100
_B = 128
_Q = 8
_CH = 10
_G = _N // (_Q * _CH)


def _body(idx_s, w_s, u_s, buf, noise, ns_row, idx_col, idx_row, u_col,
          out, new_buf, steps_out, new_ns_out, cp_v, row_v,
          sem_in, sem_out, sem_b):
    def _in_copy(g, q, slot):
        return pltpu.make_async_copy(
            buf.at[pl.ds(g * _Q * _CH + q * _CH, _CH)],
            cp_v.at[slot, q], sem_in.at[q])

    def _out_copy(g, q, slot):
        return pltpu.make_async_copy(
            cp_v.at[slot, q],
            new_buf.at[pl.ds(g * _Q * _CH + q * _CH, _CH)], sem_out.at[q])

    def copy_pair(p, carry):
        g0 = 2 * p
        g1 = 2 * p + 1
        for q in range(_Q):
            _in_copy(g0, q, 0).start()
        for q in range(_Q):
            _in_copy(g1, q, 1).start()
        for q in range(_Q):
            _in_copy(g0, q, 0).wait()
        for q in range(_Q):
            _out_copy(g0, q, 0).start()
        for q in range(_Q):
            _in_copy(g1, q, 1).wait()
        for q in range(_Q):
            _out_copy(g1, q, 1).start()
        for q in range(_Q):
            _out_copy(g0, q, 0).wait()
        for q in range(_Q):
            _out_copy(g1, q, 1).wait()
        return carry

    ns = ns_row[...]
    ic = idx_col[...]
    ir = idx_row[...]
    rc = u_col[...] < _REINIT_P
    col_ids = jax.lax.broadcasted_iota(jnp.int32, (_B, _N), 1)
    onehot = ic == col_ids
    g = jnp.sum(jnp.where(onehot, ns, 0.0), axis=1, keepdims=True)
    steps = jnp.where(rc, 0.0, g)
    steps_out[...] = steps
    bi = jax.lax.broadcasted_iota(jnp.int32, (_B, _B), 0)
    bj = jax.lax.broadcasted_iota(jnp.int32, (_B, _B), 1)
    later_same = (ic == ir) & (bj > bi)
    winner = jnp.logical_not(jnp.any(later_same, axis=1, keepdims=True))
    sc_mask = onehot & winner
    contrib = jnp.sum(jnp.where(sc_mask, steps, 0.0), axis=0, keepdims=True)
    written = jnp.any(sc_mask, axis=0, keepdims=True)
    new_ns_out[...] = jnp.where(written, contrib, ns)

    jax.lax.fori_loop(0, _G // 2, copy_pair, 0)
    for q in range(_Q):
        _in_copy(_G - 1, q, 0).start()
    for q in range(_Q):
        _in_copy(_G - 1, q, 0).wait()
    for q in range(_Q):
        _out_copy(_G - 1, q, 0).start()
    for q in range(_Q):
        _out_copy(_G - 1, q, 0).wait()

    def gather_group(gg, carry):
        base = gg * _Q
        for q in range(_Q):
            b = base + q
            reinit = u_s[b] < _REINIT_P

            @pl.when(reinit)
            def _():
                pltpu.make_async_copy(noise.at[pl.ds(b, 1)],
                                      row_v.at[q], sem_in.at[q]).start()

            @pl.when(jnp.logical_not(reinit))
            def _():
                pltpu.make_async_copy(buf.at[pl.ds(idx_s[b], 1)],
                                      row_v.at[q], sem_in.at[q]).start()

        for q in range(_Q):
            pltpu.make_async_copy(noise.at[pl.ds(0, 1)],
                                  row_v.at[q], sem_in.at[q]).wait()
        for q in range(_Q):
            b = base + q
            pltpu.make_async_copy(row_v.at[q],
                                  out.at[pl.ds(b, 1)], sem_out.at[q]).start()
        for q in range(_Q):
            b = base + q
            pltpu.make_async_copy(row_v.at[q],
                                  out.at[pl.ds(b, 1)], sem_out.at[q]).wait()
        return carry

    jax.lax.fori_loop(0, _B // _Q, gather_group, 0)

    def scatter_one(b, carry):
        cond = (u_s[b] < _REINIT_P) & (w_s[b] == b)

        @pl.when(cond)
        def _():
            pltpu.make_async_copy(noise.at[pl.ds(b, 1)],
                                  row_v.at[0], sem_b).start()
            pltpu.make_async_copy(noise.at[pl.ds(b, 1)],
                                  row_v.at[0], sem_b).wait()
            pltpu.make_async_copy(row_v.at[0],
                                  new_buf.at[pl.ds(idx_s[b], 1)], sem_b).start()
            pltpu.make_async_copy(row_v.at[0],
                                  new_buf.at[pl.ds(idx_s[b], 1)], sem_b).wait()

        return carry

    jax.lax.fori_loop(0, _B, scatter_one, 0)


def kernel(buffer, buffer_numsteps, noise, u, idx):
    idx = idx.astype(jnp.int32)
    eq = idx[:, None] == idx[None, :]
    w = jnp.max(jnp.where(eq, jnp.arange(_B, dtype=jnp.int32)[None, :], -1), axis=1)

    smem = pltpu.MemorySpace.SMEM
    hbm = pltpu.MemorySpace.HBM
    out, new_buffer, steps, new_ns = pl.pallas_call(
        _body,
        in_specs=[
            pl.BlockSpec(memory_space=smem),
            pl.BlockSpec(memory_space=smem),
            pl.BlockSpec(memory_space=smem),
            pl.BlockSpec(memory_space=hbm),
            pl.BlockSpec(memory_space=hbm),
            pl.BlockSpec((1, _N), lambda: (0, 0)),
            pl.BlockSpec((_B, 1), lambda: (0, 0)),
            pl.BlockSpec((1, _B), lambda: (0, 0)),
            pl.BlockSpec((_B, 1), lambda: (0, 0)),
        ],
        out_specs=[
            pl.BlockSpec(memory_space=hbm),
            pl.BlockSpec(memory_space=hbm),
            pl.BlockSpec((_B, 1), lambda: (0, 0)),
            pl.BlockSpec((1, _N), lambda: (0, 0)),
        ],
        out_shape=[
            jax.ShapeDtypeStruct((_B, _R, _C), jnp.float32),
            jax.ShapeDtypeStruct((_N, _R, _C), jnp.float32),
            jax.ShapeDtypeStruct((_B, 1), jnp.float32),
            jax.ShapeDtypeStruct((1, _N), jnp.float32),
        ],
        scratch_shapes=[
            pltpu.VMEM((2, _Q, _CH, _R, _C), jnp.float32),
            pltpu.VMEM((_Q, 1, _R, _C), jnp.float32),
            pltpu.SemaphoreType.DMA((_Q,)),
            pltpu.SemaphoreType.DMA((_Q,)),
            pltpu.SemaphoreType.DMA,
        ],
    )(idx, w, u, buffer, noise, buffer_numsteps.reshape(1, _N),
      idx.reshape(_B, 1), idx.reshape(1, _B), u.reshape(_B, 1))

    return out, steps.reshape(_B), new_buffer, new_ns.reshape(_N)
